# baseline (speedup 1.0000x reference)
import jax
import jax.numpy as jnp
from jax import lax
from jax.experimental import pallas as pl
from jax.experimental.pallas import tpu as pltpu
from jax.experimental.pallas import tpu_sc as plsc

D = 64
DP = 128
SCALE = 8.0

NC = 2
NS = 16
NW = NC * NS

NB = 16384
NT = 50
G = 128
C = 256
B_PER_W = NB // NW
NCH = NT * (B_PER_W // C)

S_T = 8 * 128 * 8 * 128
S_JHI = 128 * 8 * 128
S_BHI = 8 * 128
OUT_FLAT = NT * S_T


def _body(xt_hbm, ep_hbm, out_hbm,
          idx_v, g0_v, g1_v, s0_v, s1_v,
          gsem0, gsem1, wsem0, wsem1):
    wid = lax.axis_index("s") * NC + lax.axis_index("c")
    b0 = wid * B_PER_W

    gbuf = (g0_v, g1_v)
    sbuf = (s0_v, s1_v)
    gsem = (gsem0, gsem1)
    wsem = (wsem0, wsem1)

    lane = lax.iota(jnp.int32, 16)

    pltpu.sync_copy(xt_hbm.at[:, pl.ds(b0, B_PER_W)], idx_v)

    def fire(c, bf):
        t = c // 2
        off = (c % 2) * C
        for k in range(C // G):
            pltpu.async_copy(
                ep_hbm.at[idx_v.at[t, pl.ds(off + k * G, G)]],
                gbuf[bf].at[pl.ds(k * G, G)],
                gsem[bf],
            )

    def wait_gather(bf):
        pltpu.make_async_copy(
            ep_hbm.at[pl.ds(0, C)], gbuf[bf], gsem[bf]
        ).wait()

    def transpose_scale(bf):
        g = gbuf[bf]
        s = sbuf[bf]

        @plsc.parallel_loop(0, C * D // 16, step=1, unroll=4)
        def _(v):
            j = v >> 4
            bq = v & 15
            rows = (bq << 4) + lane
            cols = lane * 0 + j
            vals = plsc.load_gather(g, [rows, cols])
            s[pl.ds((((j >> 3) << 1) + (bq >> 3)) * 1024
                    + ((j & 7) << 7) + ((bq & 7) << 4), 16)] = vals * SCALE

    def start_wb(c, bf):
        t = c // 2
        base = t * S_T + (wid * 4 + (c % 2) * 2) * S_BHI
        for jhi in range(8):
            pltpu.async_copy(
                sbuf[bf].at[pl.ds(jhi * 2048, 2048)],
                out_hbm.at[pl.ds(base + jhi * S_JHI, 2048)],
                wsem[bf],
            )

    def wait_wb(bf):
        pltpu.make_async_copy(
            sbuf[bf], out_hbm.at[pl.ds(0, C * D)], wsem[bf]
        ).wait()

    fire(0, 0)
    fire(1, 1)
    for bf in range(2):
        wait_gather(bf)
        transpose_scale(bf)
        fire(2 + bf, bf)
        start_wb(bf, bf)

    def step(o, _):
        for bf in range(2):
            c = 2 * o + bf
            wait_gather(bf)
            wait_wb(bf)
            transpose_scale(bf)
            fire(c + 2, bf)
            start_wb(c, bf)
        return 0

    lax.fori_loop(1, NCH // 2 - 1, step, 0)

    for bf in range(2):
        c = NCH - 2 + bf
        wait_gather(bf)
        wait_wb(bf)
        transpose_scale(bf)
        start_wb(c, bf)
    for bf in range(2):
        wait_wb(bf)


def kernel(x, E):
    xt = x.T.astype(jnp.int32)
    ep = jnp.pad(E, ((0, 0), (0, DP - D)))
    mesh = plsc.VectorSubcoreMesh(
        core_axis_name="c", subcore_axis_name="s", num_cores=NC, num_subcores=NS
    )
    out1 = pl.kernel(
        _body,
        out_type=jax.ShapeDtypeStruct((OUT_FLAT,), jnp.float32),
        mesh=mesh,
        scratch_types=[
            pltpu.VMEM((NT, B_PER_W), jnp.int32),
            pltpu.VMEM((C, DP), jnp.float32),
            pltpu.VMEM((C, DP), jnp.float32),
            pltpu.VMEM((C * D,), jnp.float32),
            pltpu.VMEM((C * D,), jnp.float32),
            pltpu.SemaphoreType.DMA,
            pltpu.SemaphoreType.DMA,
            pltpu.SemaphoreType.DMA,
            pltpu.SemaphoreType.DMA,
        ],
        compiler_params=pltpu.CompilerParams(
            use_tc_tiling_on_sc=False, needs_layout_passes=False
        ),
    )(xt, ep)
    out5 = out1.reshape(NT, 8, NB // G, 8, G)
    return out5.transpose(2, 4, 0, 1, 3).reshape(NB, NT, D)


# speedup vs baseline: 1.2406x; 1.2406x over previous
import jax
import jax.numpy as jnp
from jax import lax
from jax.experimental import pallas as pl
from jax.experimental.pallas import tpu as pltpu
from jax.experimental.pallas import tpu_sc as plsc

D = 64
SCALE = 8.0

NC = 2
NS = 16
NW = NC * NS

B_TOTAL = 16384 * 50
G = 128
C = 256
KG = C // G
ROWS_PER_W = B_TOTAL // NW
NCH = ROWS_PER_W // C
IDX_ROWS = ROWS_PER_W // G


def _body(x_hbm, table_hbm, out_hbm,
          idx_v, g0_v, g1_v, o0_v, o1_v,
          gsem0, gsem1, wsem0, wsem1):
    wid = lax.axis_index("s") * NC + lax.axis_index("c")
    row0 = wid * ROWS_PER_W

    gbuf = (g0_v, g1_v)
    obuf = (o0_v, o1_v)
    gsem = (gsem0, gsem1)
    wsem = (wsem0, wsem1)

    pltpu.sync_copy(x_hbm.at[pl.ds(wid * IDX_ROWS, IDX_ROWS)], idx_v)

    def fire(c, b):
        for k in range(KG):
            pltpu.async_copy(
                table_hbm.at[idx_v.at[c * KG + k]],
                gbuf[b].at[pl.ds(k * G, G)],
                gsem[b],
            )

    def wait_gather(b):
        pltpu.make_async_copy(
            out_hbm.at[pl.ds(0, C)], gbuf[b], gsem[b]
        ).wait()

    def scale(b):
        @plsc.parallel_loop(0, C, step=1, unroll=4)
        def _(r):
            for j in range(D // 16):
                obuf[b][r, pl.ds(j * 16, 16)] = (
                    gbuf[b][r, pl.ds(j * 16, 16)] * SCALE
                )

    def start_wb(c, b):
        pltpu.async_copy(obuf[b], out_hbm.at[pl.ds(row0 + c * C, C)], wsem[b])

    def wait_wb(b):
        pltpu.make_async_copy(obuf[b], out_hbm.at[pl.ds(0, C)], wsem[b]).wait()

    fire(0, 0)
    fire(1, 1)
    for b in range(2):
        wait_gather(b)
        scale(b)
        fire(2 + b, b)
        start_wb(b, b)

    def step(o, _):
        for b in range(2):
            c = 2 * o + b
            wait_gather(b)
            wait_wb(b)
            scale(b)
            fire(c + 2, b)
            start_wb(c, b)
        return 0

    lax.fori_loop(1, NCH // 2 - 1, step, 0)

    for b in range(2):
        c = NCH - 2 + b
        wait_gather(b)
        wait_wb(b)
        scale(b)
        start_wb(c, b)
    for b in range(2):
        wait_wb(b)


def kernel(x, E):
    x_flat = x.reshape(B_TOTAL // G, G).astype(jnp.int32)
    mesh = plsc.VectorSubcoreMesh(
        core_axis_name="c", subcore_axis_name="s", num_cores=NC, num_subcores=NS
    )
    out = pl.kernel(
        _body,
        out_type=jax.ShapeDtypeStruct((B_TOTAL, D), jnp.float32),
        mesh=mesh,
        scratch_types=[
            pltpu.VMEM((IDX_ROWS, G), jnp.int32),
            pltpu.VMEM((C, D), jnp.float32),
            pltpu.VMEM((C, D), jnp.float32),
            pltpu.VMEM((C, D), jnp.float32),
            pltpu.VMEM((C, D), jnp.float32),
            pltpu.SemaphoreType.DMA,
            pltpu.SemaphoreType.DMA,
            pltpu.SemaphoreType.DMA,
            pltpu.SemaphoreType.DMA,
        ],
        compiler_params=pltpu.CompilerParams(use_tc_tiling_on_sc=False),
    )(x_flat, E)
    return out.reshape(x.shape[0], x.shape[1], D)
